# Initial kernel scaffold; baseline (speedup 1.0000x reference)
import jax
import jax.numpy as jnp
from jax.experimental import pallas as pl


def kernel(tokenized_prompts, token_embedding, ctx_pos, ctx_neg):
    raise NotImplementedError("write your pallas kernel here")


# speedup vs baseline: 1.3865x; 1.3865x over previous
import functools

import jax
import jax.numpy as jnp
from jax import lax
from jax.experimental import pallas as pl
from jax.experimental.pallas import tpu as pltpu
from jax.experimental.pallas import tpu_sc as plsc

N_CTX = 8
CTX_LEN = 77
CTX_DIM = 768
BATCH = 1024
NUM_WORKERS = 32
B_PER_W = BATCH // NUM_WORKERS


N_SUF = CTX_LEN - 1 - N_CTX


def _prompt_body(tok_hbm, sos_hbm, suf_hbm, table_hbm, ctxp_hbm, ctxn_hbm,
                 out_hbm, tokout_hbm,
                 tokblk_v, sos_v, suf_v, buf_v, ctxn_v, sem):
    wid = lax.axis_index("s") * 2 + lax.axis_index("c")
    base = wid * B_PER_W

    pltpu.sync_copy(ctxp_hbm.at[0], buf_v.at[pl.ds(1, N_CTX)])
    pltpu.sync_copy(ctxn_hbm.at[0], ctxn_v)
    pltpu.sync_copy(tok_hbm.at[pl.ds(base, B_PER_W)], tokblk_v)
    pltpu.sync_copy(sos_hbm.at[pl.ds(base, B_PER_W)], sos_v)
    pltpu.sync_copy(suf_hbm.at[pl.ds(base, B_PER_W)], suf_v)

    pltpu.sync_copy(tokblk_v, tokout_hbm.at[pl.ds(base, B_PER_W)])
    pltpu.sync_copy(tokblk_v, tokout_hbm.at[pl.ds(base + BATCH, B_PER_W)])

    def body(i, carry):
        b = base + i
        c0 = pltpu.async_copy(table_hbm.at[sos_v.at[i]],
                              buf_v.at[pl.ds(0, 1)], sem)
        c1 = pltpu.async_copy(table_hbm.at[suf_v.at[i]],
                              buf_v.at[pl.ds(1 + N_CTX, N_SUF)], sem)
        c0.wait()
        c1.wait()
        pltpu.sync_copy(buf_v, out_hbm.at[b])
        pltpu.sync_copy(buf_v.at[pl.ds(0, 1)],
                        out_hbm.at[b + BATCH, pl.ds(0, 1)])
        pltpu.sync_copy(ctxn_v, out_hbm.at[b + BATCH, pl.ds(1, N_CTX)])
        pltpu.sync_copy(buf_v.at[pl.ds(1 + N_CTX, N_SUF)],
                        out_hbm.at[b + BATCH, pl.ds(1 + N_CTX, N_SUF)])
        return carry

    lax.fori_loop(0, B_PER_W, body, 0)


def kernel(tokenized_prompts, token_embedding, ctx_pos, ctx_neg):
    mesh = plsc.VectorSubcoreMesh(core_axis_name="c", subcore_axis_name="s")
    f = functools.partial(
        pl.kernel,
        mesh=mesh,
        compiler_params=pltpu.CompilerParams(use_tc_tiling_on_sc=False),
        out_type=(
            jax.ShapeDtypeStruct((2 * BATCH, CTX_LEN, CTX_DIM), jnp.float32),
            jax.ShapeDtypeStruct((2 * BATCH, CTX_LEN), jnp.int32),
        ),
        scratch_types=[
            pltpu.VMEM((B_PER_W, CTX_LEN), jnp.int32),
            pltpu.VMEM((B_PER_W, 1), jnp.int32),
            pltpu.VMEM((B_PER_W, N_SUF), jnp.int32),
            pltpu.VMEM((CTX_LEN, CTX_DIM), jnp.float32),
            pltpu.VMEM((N_CTX, CTX_DIM), jnp.float32),
            pltpu.SemaphoreType.DMA,
        ],
    )(_prompt_body)
    sos_idx = tokenized_prompts[:, :1]
    suf_idx = tokenized_prompts[:, 1 + N_CTX:]
    return f(tokenized_prompts, sos_idx, suf_idx,
             token_embedding, ctx_pos, ctx_neg)
